# baseline (speedup 1.0000x reference)
import functools

import jax
import jax.numpy as jnp
from jax import lax
from jax.experimental import pallas as pl
from jax.experimental.pallas import tpu as pltpu
from jax.experimental.pallas import tpu_sc as plsc

N = 1_000_000
D = 64
B = 98304
TWOB = 2 * B
NLAYER = 5

NC = 2
NS = 16
NW = NC * NS
PER_W = TWOB // NW
CHUNK = 128
NCHUNK = PER_W // CHUNK
GCH = 64
NCHUNK2 = PER_W // GCH

BLK = 2048
NBLK = B // BLK
PACK_R = 20000
PACK_G = N // PACK_R


def _sc_gather_fn():
    mesh = plsc.VectorSubcoreMesh(core_axis_name="c", subcore_axis_name="s")

    @functools.partial(
        pl.kernel,
        out_type=jax.ShapeDtypeStruct((B, 2 * D), jnp.float32),
        mesh=mesh,
        compiler_params=pltpu.CompilerParams(
            use_tc_tiling_on_sc=True, needs_layout_passes=False),
        scratch_types=[
            pltpu.VMEM((GCH,), jnp.int32),
            pltpu.VMEM((GCH * 8, D), jnp.float32),
            pltpu.VMEM((GCH // 2, 2 * D), jnp.float32),
            pltpu.SemaphoreType.DMA,
            pltpu.SemaphoreType.DMA,
        ],
    )
    def sc_gather(u_hbm, table_hbm, out_hbm, idx_v, stage, rows2,
                  sem_a, sem_b):
        wid = lax.axis_index("s") * NC + lax.axis_index("c")
        rowbase2 = wid * (PER_W // 2)
        iota16 = lax.iota(jnp.int32, 16)

        def step(i, _):
            pltpu.sync_copy(u_hbm.at[wid * NCHUNK2 + i], idx_v)

            def fire_grp(gi, _):
                v = idx_v[pl.ds(gi * 16, 16)]
                gv = (v >> 3) * 8
                for l in range(16):
                    k = gi * 16 + l
                    pltpu.async_copy(
                        table_hbm.at[pl.ds(pl.multiple_of(gv[l], 8), 8)],
                        stage.at[pl.ds(k * 8, 8)], sem_a)
                return 0

            lax.fori_loop(0, GCH // 16, fire_grp, 0)
            pltpu.make_async_copy(
                table_hbm.at[pl.ds(0, GCH * 8)], stage, sem_a).wait()

            def ext_grp(gi, _):
                v = idx_v[pl.ds(gi * 16, 16)]
                rv = v & 7
                for l in range(0, 16, 2):
                    p = gi * 8 + l // 2
                    ra = (gi * 16 + l) * 8 + rv[l]
                    rb = (gi * 16 + l + 1) * 8 + rv[l + 1]
                    for g4 in range(4):
                        cols = iota16 + 16 * g4
                        va = plsc.load_gather(
                            stage, [jnp.full((16,), ra, jnp.int32), cols])
                        rows2[p, pl.ds(16 * g4, 16)] = va
                        vb = plsc.load_gather(
                            stage, [jnp.full((16,), rb, jnp.int32), cols])
                        rows2[p, pl.ds(D + 16 * g4, 16)] = vb
                return 0

            lax.fori_loop(0, GCH // 16, ext_grp, 0)
            pltpu.sync_copy(
                rows2,
                out_hbm.at[pl.ds(rowbase2 + i * (GCH // 2), GCH // 2)])
            return 0

        lax.fori_loop(0, NCHUNK2, step, 0)

    return sc_gather


def _tc_loss_body(g_ref, lab_ref, lay_ref, w_ref, lt_ref, q_ref, acc_ref):
    blk = g_ref[...]
    ei = blk[:, :D]
    ej = blk[:, D:]
    x = jnp.dot(ei, w_ref[...], preferred_element_type=jnp.float32)
    y = jnp.dot(ej, w_ref[...], preferred_element_type=jnp.float32)
    rxy = jnp.sum(x * y, axis=1, keepdims=True)
    s1 = jnp.dot(x + y, lt_ref[...], preferred_element_type=jnp.float32)
    lay = lay_ref[...]
    onehot = (lay == lax.broadcasted_iota(jnp.int32, (BLK, 8), 1)).astype(jnp.float32)
    inner = rxy + jnp.sum(onehot * (s1 + q_ref[...]), axis=1, keepdims=True)
    t = lab_ref[...] * inner
    part = jnp.sum(jax.nn.log_sigmoid(t))

    @pl.when(pl.program_id(0) == 0)
    def _():
        acc_ref[0, 0] = 0.0

    acc_ref[0, 0] += -part


def kernel(u_i, u_j, this_layer, label, embedding, L_embedding, W):
    m = lax.iota(jnp.int32, TWOB)
    u_all = jnp.where(
        m % 2 == 0,
        jnp.repeat(u_i.astype(jnp.int32), 2),
        jnp.repeat(u_j.astype(jnp.int32), 2),
    ).reshape(TWOB // GCH, GCH)

    g2 = _sc_gather_fn()(u_all, embedding)

    lab = label.astype(jnp.float32).reshape(B, 1)
    lay = this_layer.astype(jnp.int32).reshape(B, 1)
    lt = jnp.zeros((D, 8), jnp.float32).at[:, :NLAYER].set(L_embedding.T)
    q = jnp.zeros((1, 8), jnp.float32).at[0, :NLAYER].set(
        jnp.sum(L_embedding * L_embedding, axis=1))

    loss = pl.pallas_call(
        _tc_loss_body,
        grid=(NBLK,),
        in_specs=[
            pl.BlockSpec((BLK, 2 * D), lambda i: (i, 0)),
            pl.BlockSpec((BLK, 1), lambda i: (i, 0)),
            pl.BlockSpec((BLK, 1), lambda i: (i, 0)),
            pl.BlockSpec((D, D), lambda i: (0, 0)),
            pl.BlockSpec((D, 8), lambda i: (0, 0)),
            pl.BlockSpec((1, 8), lambda i: (0, 0)),
        ],
        out_specs=pl.BlockSpec(memory_space=pltpu.SMEM),
        out_shape=jax.ShapeDtypeStruct((1, 1), jnp.float32),
    )(g2, lab, lay, W, lt, q)
    return loss[0, 0]


# speedup vs baseline: 1.1196x; 1.1196x over previous
import functools

import jax
import jax.numpy as jnp
from jax import lax
from jax.experimental import pallas as pl
from jax.experimental.pallas import tpu as pltpu
from jax.experimental.pallas import tpu_sc as plsc

N = 1_000_000
D = 64
B = 98304
TWOB = 2 * B
NLAYER = 5

NC = 2
NS = 16
NW = NC * NS
PER_W = TWOB // NW
CHUNK = 128
NCHUNK = PER_W // CHUNK
GCH = 32
NCHUNK2 = PER_W // GCH

BLK = 2048
NBLK = B // BLK
PACK_R = 20000
PACK_G = N // PACK_R


def _sc_gather_fn():
    mesh = plsc.VectorSubcoreMesh(core_axis_name="c", subcore_axis_name="s")

    @functools.partial(
        pl.kernel,
        out_type=jax.ShapeDtypeStruct((B, 2 * D), jnp.float32),
        mesh=mesh,
        compiler_params=pltpu.CompilerParams(
            use_tc_tiling_on_sc=True, needs_layout_passes=False),
        scratch_types=[
            pltpu.VMEM((NCHUNK2, GCH), jnp.int32),
            pltpu.VMEM((GCH * 8, D), jnp.float32),
            pltpu.VMEM((GCH * 8, D), jnp.float32),
            pltpu.VMEM((GCH // 2, 2 * D), jnp.float32),
            pltpu.VMEM((GCH // 2, 2 * D), jnp.float32),
            pltpu.SemaphoreType.DMA,
            pltpu.SemaphoreType.DMA,
            pltpu.SemaphoreType.DMA,
            pltpu.SemaphoreType.DMA,
        ],
    )
    def sc_gather(u_hbm, table_hbm, out_hbm, idx_v, stage_a, stage_b,
                  rows_a, rows_b, sg_a, sg_b, so_a, so_b):
        wid = lax.axis_index("s") * NC + lax.axis_index("c")
        rowbase2 = wid * (PER_W // 2)
        iota16 = lax.iota(jnp.int32, 16)

        pltpu.sync_copy(u_hbm.at[pl.ds(wid * NCHUNK2, NCHUNK2)], idx_v)

        def fire(i, stage, sem):
            def fire_grp(gi, _):
                v = idx_v[i, pl.ds(gi * 16, 16)]
                gv = (v >> 3) * 8
                for l in range(16):
                    k = gi * 16 + l
                    pltpu.async_copy(
                        table_hbm.at[pl.ds(pl.multiple_of(gv[l], 8), 8)],
                        stage.at[pl.ds(k * 8, 8)], sem)
                return 0

            lax.fori_loop(0, GCH // 16, fire_grp, 0)

        def drain_gather(stage, sem):
            pltpu.make_async_copy(
                table_hbm.at[pl.ds(0, GCH * 8)], stage, sem).wait()

        def extract(i, stage, rows2):
            def ext_grp(gi, _):
                v = idx_v[i, pl.ds(gi * 16, 16)]
                rv = v & 7
                for l in range(0, 16, 2):
                    p = gi * 8 + l // 2
                    ra = (gi * 16 + l) * 8 + rv[l]
                    rb = (gi * 16 + l + 1) * 8 + rv[l + 1]
                    for g4 in range(4):
                        cols = iota16 + 16 * g4
                        va = plsc.load_gather(
                            stage, [jnp.full((16,), ra, jnp.int32), cols])
                        rows2[p, pl.ds(16 * g4, 16)] = va
                        vb = plsc.load_gather(
                            stage, [jnp.full((16,), rb, jnp.int32), cols])
                        rows2[p, pl.ds(D + 16 * g4, 16)] = vb
                return 0

            lax.fori_loop(0, GCH // 16, ext_grp, 0)

        def out_slice(i):
            return out_hbm.at[pl.ds(rowbase2 + i * (GCH // 2), GCH // 2)]

        def drain_out(i, rows2, sem):
            pltpu.make_async_copy(rows2, out_slice(i), sem).wait()

        fire(0, stage_a, sg_a)
        fire(1, stage_b, sg_b)

        def step(j, _):
            ia = 2 * j
            ib = 2 * j + 1
            drain_gather(stage_a, sg_a)

            @pl.when(ia >= 2)
            def _():
                drain_out(ia - 2, rows_a, so_a)

            extract(ia, stage_a, rows_a)
            pltpu.async_copy(rows_a, out_slice(ia), so_a)

            @pl.when(ia + 2 < NCHUNK2)
            def _():
                fire(ia + 2, stage_a, sg_a)

            drain_gather(stage_b, sg_b)

            @pl.when(ib >= 2)
            def _():
                drain_out(ib - 2, rows_b, so_b)

            extract(ib, stage_b, rows_b)
            pltpu.async_copy(rows_b, out_slice(ib), so_b)

            @pl.when(ib + 2 < NCHUNK2)
            def _():
                fire(ib + 2, stage_b, sg_b)

            return 0

        lax.fori_loop(0, NCHUNK2 // 2, step, 0)
        drain_out(NCHUNK2 - 2, rows_a, so_a)
        drain_out(NCHUNK2 - 1, rows_b, so_b)

    return sc_gather


def _tc_loss_body(g_ref, lab_ref, lay_ref, w_ref, lt_ref, q_ref, acc_ref):
    blk = g_ref[...]
    ei = blk[:, :D]
    ej = blk[:, D:]
    x = jnp.dot(ei, w_ref[...], preferred_element_type=jnp.float32)
    y = jnp.dot(ej, w_ref[...], preferred_element_type=jnp.float32)
    rxy = jnp.sum(x * y, axis=1, keepdims=True)
    s1 = jnp.dot(x + y, lt_ref[...], preferred_element_type=jnp.float32)
    lay = lay_ref[...]
    onehot = (lay == lax.broadcasted_iota(jnp.int32, (BLK, 8), 1)).astype(jnp.float32)
    inner = rxy + jnp.sum(onehot * (s1 + q_ref[...]), axis=1, keepdims=True)
    t = lab_ref[...] * inner
    part = jnp.sum(jax.nn.log_sigmoid(t))

    @pl.when(pl.program_id(0) == 0)
    def _():
        acc_ref[0, 0] = 0.0

    acc_ref[0, 0] += -part


def kernel(u_i, u_j, this_layer, label, embedding, L_embedding, W):
    m = lax.iota(jnp.int32, TWOB)
    u_all = jnp.where(
        m % 2 == 0,
        jnp.repeat(u_i.astype(jnp.int32), 2),
        jnp.repeat(u_j.astype(jnp.int32), 2),
    ).reshape(TWOB // GCH, GCH)

    g2 = _sc_gather_fn()(u_all, embedding)

    lab = label.astype(jnp.float32).reshape(B, 1)
    lay = this_layer.astype(jnp.int32).reshape(B, 1)
    lt = jnp.zeros((D, 8), jnp.float32).at[:, :NLAYER].set(L_embedding.T)
    q = jnp.zeros((1, 8), jnp.float32).at[0, :NLAYER].set(
        jnp.sum(L_embedding * L_embedding, axis=1))

    loss = pl.pallas_call(
        _tc_loss_body,
        grid=(NBLK,),
        in_specs=[
            pl.BlockSpec((BLK, 2 * D), lambda i: (i, 0)),
            pl.BlockSpec((BLK, 1), lambda i: (i, 0)),
            pl.BlockSpec((BLK, 1), lambda i: (i, 0)),
            pl.BlockSpec((D, D), lambda i: (0, 0)),
            pl.BlockSpec((D, 8), lambda i: (0, 0)),
            pl.BlockSpec((1, 8), lambda i: (0, 0)),
        ],
        out_specs=pl.BlockSpec(memory_space=pltpu.SMEM),
        out_shape=jax.ShapeDtypeStruct((1, 1), jnp.float32),
    )(g2, lab, lay, W, lt, q)
    return loss[0, 0]


# speedup vs baseline: 1.2109x; 1.0815x over previous
import functools

import jax
import jax.numpy as jnp
from jax import lax
from jax.experimental import pallas as pl
from jax.experimental.pallas import tpu as pltpu
from jax.experimental.pallas import tpu_sc as plsc

N = 1_000_000
D = 64
B = 98304
TWOB = 2 * B
NLAYER = 5

NC = 2
NS = 16
NW = NC * NS
PER_W = TWOB // NW
CHUNK = 128
NCHUNK = PER_W // CHUNK

BLK = 2048
NBLK = B // BLK


def _sc_gather_fn():
    mesh = plsc.VectorSubcoreMesh(core_axis_name="c", subcore_axis_name="s")

    @functools.partial(
        pl.kernel,
        out_type=jax.ShapeDtypeStruct((TWOB, D), jnp.float32),
        mesh=mesh,
        compiler_params=pltpu.CompilerParams(use_tc_tiling_on_sc=False),
        scratch_types=[
            pltpu.VMEM((NCHUNK, CHUNK), jnp.int32),
            pltpu.VMEM((CHUNK, D), jnp.float32),
            pltpu.VMEM((CHUNK, D), jnp.float32),
            pltpu.SemaphoreType.DMA,
            pltpu.SemaphoreType.DMA,
        ],
    )
    def sc_gather(u_hbm, table_hbm, out_hbm, idx_v, rows_a, rows_b, sem_a, sem_b):
        wid = lax.axis_index("s") * NC + lax.axis_index("c")
        rowbase = wid * PER_W
        pltpu.sync_copy(u_hbm.at[pl.ds(wid * NCHUNK, NCHUNK)], idx_v)

        def step(i, _):
            c0 = 2 * i
            cp_a = pltpu.async_copy(table_hbm.at[idx_v.at[c0]], rows_a, sem_a)
            cp_b = pltpu.async_copy(table_hbm.at[idx_v.at[c0 + 1]], rows_b, sem_b)
            cp_a.wait()
            pltpu.sync_copy(rows_a, out_hbm.at[pl.ds(rowbase + c0 * CHUNK, CHUNK)])
            cp_b.wait()
            pltpu.sync_copy(rows_b, out_hbm.at[pl.ds(rowbase + (c0 + 1) * CHUNK, CHUNK)])
            return 0

        lax.fori_loop(0, NCHUNK // 2, step, 0)

    return sc_gather


def _tc_loss_body(g_ref, lab_ref, lay_ref, w_ref, lt_ref, q_ref, acc_ref):
    blk = g_ref[...]
    ei = blk[:, :D]
    ej = blk[:, D:]
    x = jnp.dot(ei, w_ref[...], preferred_element_type=jnp.float32)
    y = jnp.dot(ej, w_ref[...], preferred_element_type=jnp.float32)
    rxy = jnp.sum(x * y, axis=1, keepdims=True)
    s1 = jnp.dot(x + y, lt_ref[...], preferred_element_type=jnp.float32)
    lay = lay_ref[...]
    onehot = (lay == lax.broadcasted_iota(jnp.int32, (BLK, 8), 1)).astype(jnp.float32)
    inner = rxy + jnp.sum(onehot * (s1 + q_ref[...]), axis=1, keepdims=True)
    t = lab_ref[...] * inner
    part = jnp.sum(jax.nn.log_sigmoid(t))

    @pl.when(pl.program_id(0) == 0)
    def _():
        acc_ref[0, 0] = 0.0

    acc_ref[0, 0] += -part


def kernel(u_i, u_j, this_layer, label, embedding, L_embedding, W):
    m = lax.iota(jnp.int32, TWOB)
    u_all = jnp.where(
        m % 2 == 0,
        jnp.repeat(u_i.astype(jnp.int32), 2),
        jnp.repeat(u_j.astype(jnp.int32), 2),
    ).reshape(TWOB // CHUNK, CHUNK)

    gathered = _sc_gather_fn()(u_all, embedding)
    g2 = gathered.reshape(B, 2 * D)

    lab = label.astype(jnp.float32).reshape(B, 1)
    lay = this_layer.astype(jnp.int32).reshape(B, 1)
    lt = jnp.zeros((D, 8), jnp.float32).at[:, :NLAYER].set(L_embedding.T)
    q = jnp.zeros((1, 8), jnp.float32).at[0, :NLAYER].set(
        jnp.sum(L_embedding * L_embedding, axis=1))

    loss = pl.pallas_call(
        _tc_loss_body,
        grid=(NBLK,),
        in_specs=[
            pl.BlockSpec((BLK, 2 * D), lambda i: (i, 0)),
            pl.BlockSpec((BLK, 1), lambda i: (i, 0)),
            pl.BlockSpec((BLK, 1), lambda i: (i, 0)),
            pl.BlockSpec((D, D), lambda i: (0, 0)),
            pl.BlockSpec((D, 8), lambda i: (0, 0)),
            pl.BlockSpec((1, 8), lambda i: (0, 0)),
        ],
        out_specs=pl.BlockSpec(memory_space=pltpu.SMEM),
        out_shape=jax.ShapeDtypeStruct((1, 1), jnp.float32),
    )(g2, lab, lay, W, lt, q)
    return loss[0, 0]
